# baseline (speedup 1.0000x reference)
import jax
import jax.numpy as jnp
from jax.experimental import pallas as pl

_CODES = 512
_NC = 128
_LB = 2048


def _vq_body(x_ref, c_ref, o_ref):
    xb = x_ref[0, 0]
    cb = c_ref[0]
    lhs = jnp.concatenate([xb * 2.0, jnp.ones((xb.shape[0], 1), jnp.float32)], 1)
    c2 = jnp.sum(cb * cb, axis=1, keepdims=True)
    rhs = jnp.concatenate([cb, -c2], 1)
    u = jax.lax.dot_general(
        lhs, rhs,
        dimension_numbers=(((1,), (1,)), ((), ())),
        preferred_element_type=jnp.float32,
    )

    u0 = u[:, 0 * _NC:1 * _NC]
    u1 = u[:, 1 * _NC:2 * _NC]
    u2 = u[:, 2 * _NC:3 * _NC]
    u3 = u[:, 3 * _NC:4 * _NC]
    t01 = jnp.maximum(u0, u1)
    t23 = jnp.maximum(u2, u3)
    i01 = jnp.where(u1 > u0, float(1 * _NC), float(0 * _NC))
    i23 = jnp.where(u3 > u2, float(3 * _NC), float(2 * _NC))
    m = jnp.maximum(t01, t23)
    q = jnp.where(t23 > t01, i23, i01)

    maxval = jnp.max(m, axis=1, keepdims=True)
    lane = jax.lax.broadcasted_iota(jnp.int32, m.shape, 1).astype(jnp.float32)
    g = jnp.where(m == maxval, q + lane, float(_CODES))
    first = jnp.min(g, axis=1, keepdims=True)

    one = jnp.float32(1.0)
    zero = jnp.float32(0.0)
    for cidx in range(_CODES // _NC):
        o_ref[0, 0, :, cidx * _NC:(cidx + 1) * _NC] = jnp.where(
            lane == first - float(cidx * _NC), one, zero)


def kernel(x, c):
    b, h, l, d = x.shape
    s = c.shape[1]
    out = pl.pallas_call(
        _vq_body,
        grid=(b, h, l // _LB),
        in_specs=[
            pl.BlockSpec((1, 1, _LB, d), lambda i, j, k: (i, j, k, 0)),
            pl.BlockSpec((1, s, d), lambda i, j, k: (j, 0, 0)),
        ],
        out_specs=pl.BlockSpec((1, 1, _LB, s), lambda i, j, k: (i, j, k, 0)),
        out_shape=jax.ShapeDtypeStruct((b, h, l, s), jnp.float32),
    )(x, c)
    return (out, c)


# speedup vs baseline: 1.1621x; 1.1621x over previous
import jax
import jax.numpy as jnp
from jax.experimental import pallas as pl

_CODES = 512
_NC = 128
_LB = 4096


def _vq_body(x_ref, c_ref, o_ref):
    xb = x_ref[0, 0]
    cb = c_ref[0]
    lhs = jnp.concatenate([xb * 2.0, jnp.ones((xb.shape[0], 1), jnp.float32)], 1)
    c2 = jnp.sum(cb * cb, axis=1, keepdims=True)
    rhs = jnp.concatenate([cb, -c2], 1)
    u = jax.lax.dot_general(
        lhs, rhs,
        dimension_numbers=(((1,), (1,)), ((), ())),
        preferred_element_type=jnp.float32,
    )

    u0 = u[:, 0 * _NC:1 * _NC]
    u1 = u[:, 1 * _NC:2 * _NC]
    u2 = u[:, 2 * _NC:3 * _NC]
    u3 = u[:, 3 * _NC:4 * _NC]
    t01 = jnp.maximum(u0, u1)
    t23 = jnp.maximum(u2, u3)
    i01 = jnp.where(u1 > u0, float(1 * _NC), float(0 * _NC))
    i23 = jnp.where(u3 > u2, float(3 * _NC), float(2 * _NC))
    m = jnp.maximum(t01, t23)
    q = jnp.where(t23 > t01, i23, i01)

    maxval = jnp.max(m, axis=1, keepdims=True)
    lane = jax.lax.broadcasted_iota(jnp.int32, m.shape, 1).astype(jnp.float32)
    g = jnp.where(m == maxval, q + lane, float(_CODES))
    first = jnp.min(g, axis=1, keepdims=True)

    one = jnp.float32(1.0)
    zero = jnp.float32(0.0)
    for cidx in range(_CODES // _NC):
        o_ref[0, 0, :, cidx * _NC:(cidx + 1) * _NC] = jnp.where(
            lane == first - float(cidx * _NC), one, zero)


def kernel(x, c):
    b, h, l, d = x.shape
    s = c.shape[1]
    out = pl.pallas_call(
        _vq_body,
        grid=(b, h, l // _LB),
        in_specs=[
            pl.BlockSpec((1, 1, _LB, d), lambda i, j, k: (i, j, k, 0)),
            pl.BlockSpec((1, s, d), lambda i, j, k: (j, 0, 0)),
        ],
        out_specs=pl.BlockSpec((1, 1, _LB, s), lambda i, j, k: (i, j, k, 0)),
        out_shape=jax.ShapeDtypeStruct((b, h, l, s), jnp.float32),
    )(x, c)
    return (out, c)
